# baseline (speedup 1.0000x reference)
import jax
import jax.numpy as jnp
from jax.experimental import pallas as pl
from jax.experimental.pallas import tpu as pltpu

TOKENS = 8192
D_MODEL = 1024
N_EXPERTS = 16
BT = 2048
H = D_MODEL // 2


def _gate_block(xa_ref, xb_ref, w_ref, o_ref):
    wb = w_ref[...].astype(jnp.bfloat16)
    la = jnp.dot(xa_ref[...].astype(jnp.bfloat16), wb[:H],
                 preferred_element_type=jnp.float32)
    lb = jnp.dot(xb_ref[...].astype(jnp.bfloat16), wb[H:],
                 preferred_element_type=jnp.float32)
    logits = la + lb
    m = jnp.max(logits, axis=-1, keepdims=True)
    e = jnp.exp(logits - m)
    o_ref[...] = e / jnp.sum(e, axis=-1, keepdims=True)


def kernel(x, W):
    return pl.pallas_call(
        _gate_block,
        grid=(TOKENS // BT,),
        in_specs=[
            pl.BlockSpec((BT, H), lambda i: (i, 0)),
            pl.BlockSpec((BT, H), lambda i: (i, 1)),
            pl.BlockSpec((D_MODEL, N_EXPERTS), lambda i: (0, 0)),
        ],
        out_specs=pl.BlockSpec((BT, N_EXPERTS), lambda i: (i, 0)),
        out_shape=jax.ShapeDtypeStruct((TOKENS, N_EXPERTS), jnp.float32),
        compiler_params=pltpu.CompilerParams(
            dimension_semantics=("parallel",)
        ),
    )(x, x, W)
